# baseline (speedup 1.0000x reference)
import functools
import math

import jax
import jax.numpy as jnp
from jax import lax
from jax.experimental import pallas as pl
from jax.experimental.pallas import tpu as pltpu
from jax.experimental.pallas import tpu_sc as plsc

_N_GRID = 128
_NE_LVL = _N_GRID ** 3
_NC = 8
_NE = _NC * _NE_LVL
_S = _NE // 4
_MS = math.sqrt(3.0) / 1024.0
_DECAY = 0.95
_OPA = 0.01

_W = 32
_NBINS = 256
_BINSZ = _NE // _NBINS
_BINS_PER_W = _NBINS // _W
_SPW = _S // _W
_SENT = 2**31 - 1

_WS = 16384
_NWIN = _SPW // _WS
_WSP = _WS + 2048
_H1N = _W * _NWIN * _WSP

_CK = 64
_GRP = 128
_TWIN = 4096
_TABN = 1187840

_mesh = plsc.VectorSubcoreMesh(core_axis_name="c", subcore_axis_name="s",
                               num_cores=2, num_subcores=16)
_SC_PARAMS = pltpu.CompilerParams(needs_layout_passes=False)


def _wid():
    return lax.axis_index("s") * 2 + lax.axis_index("c")


def _lanes():
    return lax.iota(jnp.int32, 16)


def _v16(v):
    return pl.ds(pl.multiple_of(v * 16, 16), 16)


def _gather16(x, idxv):
    dnums = lax.GatherDimensionNumbers(
        offset_dims=(), collapsed_slice_dims=(0,), start_index_map=(0,))
    return lax.gather(x, idxv[:, None], dnums, (1,),
                      mode=lax.GatherScatterMode.PROMISE_IN_BOUNDS)


def _scal(vref, i):
    v = vref[pl.ds(pl.multiple_of((i // 16) * 16, 16), 16)]
    return jnp.sum(jnp.where(_lanes() == (i % 16), v, 0))


def _route_body(idx_hbm, den_hbm, h1i_hbm, h1v_hbm, counts_hbm,
                idxbufA, valbufA, idxbufB, valbufB, sidx, sval, hist, offs,
                cout, lsem):
    w = _wid()
    lanes = _lanes()

    def _loads(win, slot):
        woff = pl.multiple_of(w * _SPW + win * _WS, 8)
        ib = idxbufA if slot == 0 else idxbufB
        vb = valbufA if slot == 0 else valbufB
        return (pltpu.make_async_copy(idx_hbm.at[pl.ds(woff, _WS)], ib, lsem),
                pltpu.make_async_copy(den_hbm.at[pl.ds(woff, _WS)], vb, lsem))

    for cp in _loads(0, 0):
        cp.start()

    def _window(win, idxbuf, valbuf):
        def zero(i, _):
            for u in range(8):
                hist[_v16(i * 8 + u)] = jnp.zeros((16,), jnp.int32)
            return _
        lax.fori_loop(0, _NBINS // 8, zero, None)

        def fill(i, _):
            for u in range(8):
                sidx[_v16(i * 8 + u)] = jnp.full((16,), _SENT, jnp.int32)
            return _
        lax.fori_loop(0, _WSP // 128, fill, None)

        def hpass(v, _):
            iv = idxbuf[_v16(v)]
            addr = lanes * _NBINS + lax.shift_right_logical(iv, 16)
            plsc.addupdate_scatter(hist, [addr], jnp.ones((16,), jnp.int32))
            return _
        lax.fori_loop(0, _WS // 16, hpass, None)

        def prefix(bv, carry):
            bsl = pl.ds(pl.multiple_of(bv * 16, 16), 16)
            tot = jnp.zeros((16,), jnp.int32)
            for l in range(16):
                tot = tot + hist[pl.ds(pl.multiple_of(l * _NBINS + bv * 16, 16), 16)]
            padded = (tot + 7) & ~7
            incl = plsc.cumsum(padded)
            starts = carry + incl - padded
            cout[pl.ds(pl.multiple_of(win * _NBINS + bv * 16, 16), 16)] = tot
            acc = starts
            for l in range(16):
                lsl = pl.ds(pl.multiple_of(l * _NBINS + bv * 16, 16), 16)
                offs[lsl] = acc
                acc = acc + hist[lsl]
            return carry + _gather16(incl, jnp.full((16,), 15, jnp.int32))
        lax.fori_loop(0, _NBINS // 16, prefix, jnp.zeros((16,), jnp.int32))

        def spass(v, _):
            iv = idxbuf[_v16(v)]
            dv = valbuf[_v16(v)]
            addr = lanes * _NBINS + lax.shift_right_logical(iv, 16)
            dest = plsc.load_gather(offs, [addr])
            plsc.store_scatter(offs, [addr], dest + 1)
            plsc.store_scatter(sidx, [dest], iv)
            plsc.store_scatter(sval, [dest], dv)
            return _
        lax.fori_loop(0, _WS // 16, spass, None)

        hoff = pl.multiple_of((w * _NWIN + win) * _WSP, 8)
        pltpu.sync_copy(sidx, h1i_hbm.at[pl.ds(hoff, _WSP)])
        pltpu.sync_copy(sval, h1v_hbm.at[pl.ds(hoff, _WSP)])

    def pair(j, _):
        win0 = j * 2
        for cp in _loads(win0, 0):
            cp.wait()
        for cp in _loads(win0 + 1, 1):
            cp.start()
        _window(win0, idxbufA, valbufA)
        for cp in _loads(win0 + 1, 1):
            cp.wait()

        @pl.when(win0 + 2 < _NWIN)
        def _pf():
            for cp in _loads(win0 + 2, 0):
                cp.start()
        _window(win0 + 1, idxbufB, valbufB)
        return _
    lax.fori_loop(0, _NWIN // 2, pair, None)
    pltpu.sync_copy(cout, counts_hbm.at[w])


_route = functools.partial(
    pl.kernel,
    out_type=(jax.ShapeDtypeStruct((_H1N,), jnp.int32),
              jax.ShapeDtypeStruct((_H1N,), jnp.float32),
              jax.ShapeDtypeStruct((_W, _NWIN * _NBINS), jnp.int32)),
    mesh=_mesh,
    compiler_params=_SC_PARAMS,
    scratch_types=[pltpu.VMEM((_WS,), jnp.int32),
                   pltpu.VMEM((_WS,), jnp.float32),
                   pltpu.VMEM((_WS,), jnp.int32),
                   pltpu.VMEM((_WS,), jnp.float32),
                   pltpu.VMEM((_WSP,), jnp.int32),
                   pltpu.VMEM((_WSP,), jnp.float32),
                   pltpu.VMEM((_NBINS * 16,), jnp.int32),
                   pltpu.VMEM((_NBINS * 16,), jnp.int32),
                   pltpu.VMEM((_NWIN * _NBINS,), jnp.int32),
                   pltpu.SemaphoreType.DMA],
)(_route_body)


def _tabgen_body(gt_hbm, st_hbm, chs_hbm, nch_hbm, tab_hbm,
                 gv, sv, chs_v, nch_v, tabbuf):
    w = _wid()
    lanes = _lanes()
    pltpu.sync_copy(chs_hbm, chs_v)
    pltpu.sync_copy(nch_hbm, nch_v)

    def per_bin(i, _):
        b = i * _W + w
        pltpu.sync_copy(gt_hbm.at[b], gv)
        pltpu.sync_copy(st_hbm.at[b], sv)
        tstart = _scal(chs_v, b)
        n = _scal(nch_v, b)
        nw = (n + _TWIN - 1) // _TWIN

        def wnd(g, _):
            def vec(v, _):
                t = tstart + g * _TWIN + v * 16 + lanes
                r = jnp.zeros((16,), jnp.int32)
                for sz in (128, 64, 32, 16, 8, 4, 2, 1):
                    c = r + sz
                    gc = plsc.load_gather(gv, [jnp.minimum(c, 255)])
                    r = jnp.where((c < _W * _NWIN) & (gc <= t), c, r)
                g0 = plsc.load_gather(gv, [r])
                s0 = plsc.load_gather(sv, [r])
                src = s0 + _CK * (t - g0)
                tabbuf[_v16(v)] = jnp.clip(src, 0, _H1N - _CK)
                return _
            nv = (jnp.minimum(n - g * _TWIN, _TWIN) + 15) // 16
            lax.fori_loop(0, nv, vec, None)
            toff = pl.multiple_of(tstart + g * _TWIN, 8)
            pltpu.sync_copy(tabbuf, tab_hbm.at[pl.ds(toff, _TWIN)])
            return _
        lax.fori_loop(0, nw, wnd, None)
        return _
    lax.fori_loop(0, _BINS_PER_W, per_bin, None)


_tabgen = functools.partial(
    pl.kernel,
    out_type=jax.ShapeDtypeStruct((_TABN,), jnp.int32),
    mesh=_mesh,
    compiler_params=_SC_PARAMS,
    scratch_types=[pltpu.VMEM((_W * _NWIN,), jnp.int32),
                   pltpu.VMEM((_W * _NWIN,), jnp.int32),
                   pltpu.VMEM((_NBINS,), jnp.int32),
                   pltpu.VMEM((_NBINS,), jnp.int32),
                   pltpu.VMEM((_TWIN,), jnp.int32)],
)(_tabgen_body)


def _apply_body(h1i_hbm, h1v_hbm, chtab_hbm, grid_hbm, chs_hbm, nch_hbm,
                grid_out, part_out, chunk, lbi, lbv, chbuf, chs_v, nch_v,
                accv, sem):
    w = _wid()
    lanes = _lanes()
    pltpu.sync_copy(chs_hbm, chs_v)
    pltpu.sync_copy(nch_hbm, nch_v)
    accv[...] = jnp.zeros((16,), jnp.float32)

    def per_bin(i, _):
        b = i * _W + w
        chstart = _scal(chs_v, b)
        nch = _scal(nch_v, b)
        cbase = pl.multiple_of(b * _BINSZ, 8)
        pltpu.sync_copy(grid_hbm.at[pl.ds(cbase, _BINSZ)], chunk)

        def ema(v, _):
            for u in range(8):
                g = chunk[_v16(v * 8 + u)]
                chunk[_v16(v * 8 + u)] = jnp.where(g < 0.0, g, g * _DECAY)
            return _
        lax.fori_loop(0, _BINSZ // 128, ema, None)

        ngroups = (nch + _GRP - 1) // _GRP

        def group(g, _):
            coff = pl.multiple_of(chstart + g * _GRP, 8)
            pltpu.sync_copy(chtab_hbm.at[pl.ds(coff, _GRP)], chbuf)
            rem = nch - g * _GRP

            nmin = jnp.minimum(rem, _GRP)

            def issue(k, _):
                src = pl.multiple_of(_scal(chbuf, k), 8)
                dst = pl.multiple_of(k * _CK, 8)
                pltpu.make_async_copy(
                    h1i_hbm.at[pl.ds(src, _CK)],
                    lbi.at[pl.ds(dst, _CK)], sem).start()
                pltpu.make_async_copy(
                    h1v_hbm.at[pl.ds(src, _CK)],
                    lbv.at[pl.ds(dst, _CK)], sem).start()
                return _
            lax.fori_loop(0, nmin, issue, None)

            @pl.when(nmin == _GRP)
            def _bulk_drain():
                pltpu.make_async_copy(
                    h1i_hbm.at[pl.ds(0, _GRP * _CK)], lbi, sem).wait()
                pltpu.make_async_copy(
                    h1v_hbm.at[pl.ds(0, _GRP * _CK)], lbv, sem).wait()

            @pl.when(nmin < _GRP)
            def _tail_drain():
                def drain(k, _):
                    src = pl.multiple_of(_scal(chbuf, k), 8)
                    dst = pl.multiple_of(k * _CK, 8)
                    pltpu.make_async_copy(
                        h1i_hbm.at[pl.ds(src, _CK)],
                        lbi.at[pl.ds(dst, _CK)], sem).wait()
                    pltpu.make_async_copy(
                        h1v_hbm.at[pl.ds(src, _CK)],
                        lbv.at[pl.ds(dst, _CK)], sem).wait()
                    return _
                lax.fori_loop(0, nmin, drain, None)

            vcnt = jnp.minimum(rem, _GRP) * _CK

            def vec(v, _):
                pos = v * 16 + lanes
                iv = lbi[_v16(v)]
                vv = lbv[_v16(v)] * _MS
                inb = (pos < vcnt) & (iv >= cbase) & (iv < cbase + _BINSZ)
                iv = jnp.where(inb, iv, _SENT)
                vv = jnp.where(inb, vv, -1.0)
                si, sv = plsc.sort_key_val(iv, vv)
                nxt = _gather16(si, jnp.minimum(lanes + 1, 15))
                anydup = jnp.any((si == nxt) & (lanes < 15))

                def segmax(sv):
                    for s in (1, 2, 4, 8):
                        pi = _gather16(si, jnp.maximum(lanes - s, 0))
                        pv = _gather16(sv, jnp.maximum(lanes - s, 0))
                        take = (pi == si) & (lanes >= s)
                        sv = jnp.where(take, jnp.maximum(sv, pv), sv)
                    return sv
                sv = lax.cond(anydup, segmax, lambda x: x, sv)
                last = (si != nxt) | (lanes == 15)
                valid = si != _SENT
                local = jnp.clip(si - cbase, 0, _BINSZ - 1)
                wm = last & valid
                cur = plsc.load_gather(chunk, [local], mask=wm)
                upd = jnp.maximum(cur, sv)
                wm = wm & (cur >= 0.0)
                plsc.store_scatter(chunk, [local], upd, mask=wm)
                return _
            lax.fori_loop(0, (vcnt + 15) // 16, vec, None)
            return _
        lax.fori_loop(0, ngroups, group, None)

        pltpu.sync_copy(chunk, grid_out.at[pl.ds(cbase, _BINSZ)])

        @pl.when(b < _NE_LVL // _BINSZ)
        def _mean():
            def acc(v, a):
                for u in range(8):
                    a = a + jnp.maximum(chunk[_v16(v * 8 + u)], 0.0)
                return a
            accv[...] = accv[...] + lax.fori_loop(
                0, _BINSZ // 128, acc, jnp.zeros((16,), jnp.float32))
        return _
    lax.fori_loop(0, _BINS_PER_W, per_bin, None)
    pltpu.sync_copy(accv, part_out.at[w])


_apply = functools.partial(
    pl.kernel,
    out_type=(jax.ShapeDtypeStruct((_NE,), jnp.float32),
              jax.ShapeDtypeStruct((_W, 16), jnp.float32)),
    mesh=_mesh,
    compiler_params=_SC_PARAMS,
    scratch_types=[pltpu.VMEM((_BINSZ,), jnp.float32),
                   pltpu.VMEM((_GRP * _CK,), jnp.int32),
                   pltpu.VMEM((_GRP * _CK,), jnp.float32),
                   pltpu.VMEM((_GRP,), jnp.int32),
                   pltpu.VMEM((_NBINS,), jnp.int32),
                   pltpu.VMEM((_NBINS,), jnp.int32),
                   pltpu.VMEM((16,), jnp.float32),
                   pltpu.SemaphoreType.DMA],
)(_apply_body)


def _bitfield_body(thres_ref, g_ref, out_ref):
    t = thres_ref[0, 0]
    x = g_ref[...]
    bits = (x > t).astype(jnp.float32)
    l = lax.broadcasted_iota(jnp.int32, (128, 16), 0)
    k = lax.broadcasted_iota(jnp.int32, (128, 16), 1)
    w = jnp.where(l // 8 == k, jnp.exp2((l % 8).astype(jnp.float32)), 0.0)
    packed = lax.dot_general(bits, w, (((1,), (0,)), ((), ())),
                             preferred_element_type=jnp.float32)
    out_ref[...] = packed.astype(jnp.uint8)


def _bitfield(new_grid, thres):
    BLK = 2048
    rows = _NE // 128
    nblk = rows // BLK
    g2 = new_grid.reshape(rows, 128)
    out = pl.pallas_call(
        _bitfield_body,
        grid=(nblk,),
        in_specs=[pl.BlockSpec(memory_space=pltpu.SMEM),
                  pl.BlockSpec((BLK, 128), lambda i: (i, 0))],
        out_specs=pl.BlockSpec((BLK, 16), lambda i: (i, 0)),
        out_shape=jax.ShapeDtypeStruct((rows, 16), jnp.uint8),
    )(thres.reshape(1, 1), g2)
    return out.reshape(-1)


def kernel(density, idx_sample, density_grid):
    h1i, h1v, counts = _route(idx_sample, density)

    c = counts.reshape(_W, _NWIN, _NBINS)
    padded = (c + 7) & ~7
    instart = jnp.cumsum(padded, axis=2) - padded
    base = ((jnp.arange(_W, dtype=jnp.int32)[:, None, None] * _NWIN
             + jnp.arange(_NWIN, dtype=jnp.int32)[None, :, None]) * _WSP)
    srcstart = base + instart
    p_t = padded.transpose(2, 0, 1).reshape(_NBINS, _W * _NWIN)
    s_t = srcstart.transpose(2, 0, 1).reshape(_NBINS, _W * _NWIN)
    L = (p_t + _CK - 1) // _CK
    nch = L.sum(axis=1).astype(jnp.int32)
    nchr = (nch + _TWIN - 1) // _TWIN * _TWIN
    chstart = (jnp.cumsum(nchr) - nchr).astype(jnp.int32)
    rp = (jnp.cumsum(L, axis=1) - L).astype(jnp.int32)
    gt = chstart[:, None] + rp
    chtab = _tabgen(gt, s_t.astype(jnp.int32), chstart, nch)

    new_grid, partials = _apply(h1i, h1v, chtab, density_grid, chstart, nch)
    mean = partials.sum() / jnp.float32(_NE_LVL)
    thres = jnp.minimum(jnp.float32(_OPA), mean)
    return new_grid, _bitfield(new_grid, thres)


# speedup vs baseline: 1.0055x; 1.0055x over previous
import functools
import math

import jax
import jax.numpy as jnp
from jax import lax
from jax.experimental import pallas as pl
from jax.experimental.pallas import tpu as pltpu
from jax.experimental.pallas import tpu_sc as plsc

_N_GRID = 128
_NE_LVL = _N_GRID ** 3
_NC = 8
_NE = _NC * _NE_LVL
_S = _NE // 4
_MS = math.sqrt(3.0) / 1024.0
_DECAY = 0.95
_OPA = 0.01

_W = 32
_NBINS = 256
_BINSZ = _NE // _NBINS
_BINS_PER_W = _NBINS // _W
_SPW = _S // _W
_SENT = 2**31 - 1

_WS = 16384
_NWIN = _SPW // _WS
_WSP = _WS + 2048
_H1N = _W * _NWIN * _WSP

_CK = 64
_GRP = 128
_TWIN = 4096
_TABN = 1187840

_mesh = plsc.VectorSubcoreMesh(core_axis_name="c", subcore_axis_name="s",
                               num_cores=2, num_subcores=16)
_SC_PARAMS = pltpu.CompilerParams(needs_layout_passes=False)


def _wid():
    return lax.axis_index("s") * 2 + lax.axis_index("c")


def _lanes():
    return lax.iota(jnp.int32, 16)


def _v16(v):
    return pl.ds(pl.multiple_of(v * 16, 16), 16)


def _gather16(x, idxv):
    dnums = lax.GatherDimensionNumbers(
        offset_dims=(), collapsed_slice_dims=(0,), start_index_map=(0,))
    return lax.gather(x, idxv[:, None], dnums, (1,),
                      mode=lax.GatherScatterMode.PROMISE_IN_BOUNDS)


def _scal(vref, i):
    v = vref[pl.ds(pl.multiple_of((i // 16) * 16, 16), 16)]
    return jnp.sum(jnp.where(_lanes() == (i % 16), v, 0))


def _route_body(idx_hbm, den_hbm, h1i_hbm, h1v_hbm, counts_hbm,
                idxbufA, valbufA, idxbufB, valbufB, sidx, sval, hist, offs,
                cout, lsem):
    w = _wid()
    lanes = _lanes()

    def _loads(win, slot):
        woff = pl.multiple_of(w * _SPW + win * _WS, 8)
        ib = idxbufA if slot == 0 else idxbufB
        vb = valbufA if slot == 0 else valbufB
        return (pltpu.make_async_copy(idx_hbm.at[pl.ds(woff, _WS)], ib, lsem),
                pltpu.make_async_copy(den_hbm.at[pl.ds(woff, _WS)], vb, lsem))

    for cp in _loads(0, 0):
        cp.start()

    def _window(win, idxbuf, valbuf):
        def zero(i, _):
            for u in range(8):
                hist[_v16(i * 8 + u)] = jnp.zeros((16,), jnp.int32)
            return _
        lax.fori_loop(0, _NBINS // 8, zero, None)

        def fill(i, _):
            for u in range(8):
                sidx[_v16(i * 8 + u)] = jnp.full((16,), _SENT, jnp.int32)
            return _
        lax.fori_loop(0, _WSP // 128, fill, None)

        def hpass(v, _):
            iv = idxbuf[_v16(v)]
            addr = lanes * _NBINS + lax.shift_right_logical(iv, 16)
            plsc.addupdate_scatter(hist, [addr], jnp.ones((16,), jnp.int32))
            return _
        lax.fori_loop(0, _WS // 16, hpass, None)

        def prefix(bv, carry):
            bsl = pl.ds(pl.multiple_of(bv * 16, 16), 16)
            tot = jnp.zeros((16,), jnp.int32)
            for l in range(16):
                tot = tot + hist[pl.ds(pl.multiple_of(l * _NBINS + bv * 16, 16), 16)]
            padded = (tot + 7) & ~7
            incl = plsc.cumsum(padded)
            starts = carry + incl - padded
            cout[pl.ds(pl.multiple_of(win * _NBINS + bv * 16, 16), 16)] = tot
            acc = starts
            for l in range(16):
                lsl = pl.ds(pl.multiple_of(l * _NBINS + bv * 16, 16), 16)
                offs[lsl] = acc
                acc = acc + hist[lsl]
            return carry + _gather16(incl, jnp.full((16,), 15, jnp.int32))
        lax.fori_loop(0, _NBINS // 16, prefix, jnp.zeros((16,), jnp.int32))

        def spass(v, _):
            iv = idxbuf[_v16(v)]
            dv = valbuf[_v16(v)]
            addr = lanes * _NBINS + lax.shift_right_logical(iv, 16)
            dest = plsc.load_gather(offs, [addr])
            plsc.store_scatter(offs, [addr], dest + 1)
            plsc.store_scatter(sidx, [dest], iv)
            plsc.store_scatter(sval, [dest], dv)
            return _
        lax.fori_loop(0, _WS // 16, spass, None)

        hoff = pl.multiple_of((w * _NWIN + win) * _WSP, 8)
        pltpu.sync_copy(sidx, h1i_hbm.at[pl.ds(hoff, _WSP)])
        pltpu.sync_copy(sval, h1v_hbm.at[pl.ds(hoff, _WSP)])

    def pair(j, _):
        win0 = j * 2
        for cp in _loads(win0, 0):
            cp.wait()
        for cp in _loads(win0 + 1, 1):
            cp.start()
        _window(win0, idxbufA, valbufA)
        for cp in _loads(win0 + 1, 1):
            cp.wait()

        @pl.when(win0 + 2 < _NWIN)
        def _pf():
            for cp in _loads(win0 + 2, 0):
                cp.start()
        _window(win0 + 1, idxbufB, valbufB)
        return _
    lax.fori_loop(0, _NWIN // 2, pair, None)
    pltpu.sync_copy(cout, counts_hbm.at[w])


_route = functools.partial(
    pl.kernel,
    out_type=(jax.ShapeDtypeStruct((_H1N,), jnp.int32),
              jax.ShapeDtypeStruct((_H1N,), jnp.float32),
              jax.ShapeDtypeStruct((_W, _NWIN * _NBINS), jnp.int32)),
    mesh=_mesh,
    compiler_params=_SC_PARAMS,
    scratch_types=[pltpu.VMEM((_WS,), jnp.int32),
                   pltpu.VMEM((_WS,), jnp.float32),
                   pltpu.VMEM((_WS,), jnp.int32),
                   pltpu.VMEM((_WS,), jnp.float32),
                   pltpu.VMEM((_WSP,), jnp.int32),
                   pltpu.VMEM((_WSP,), jnp.float32),
                   pltpu.VMEM((_NBINS * 16,), jnp.int32),
                   pltpu.VMEM((_NBINS * 16,), jnp.int32),
                   pltpu.VMEM((_NWIN * _NBINS,), jnp.int32),
                   pltpu.SemaphoreType.DMA],
)(_route_body)


def _tabgen_body(gt_hbm, st_hbm, chs_hbm, nch_hbm, tab_hbm,
                 gv, sv, chs_v, nch_v, tabbuf):
    w = _wid()
    lanes = _lanes()
    pltpu.sync_copy(chs_hbm, chs_v)
    pltpu.sync_copy(nch_hbm, nch_v)

    def per_bin(i, _):
        b = i * _W + w
        pltpu.sync_copy(gt_hbm.at[b], gv)
        pltpu.sync_copy(st_hbm.at[b], sv)
        tstart = _scal(chs_v, b)
        n = _scal(nch_v, b)
        nw = (n + _TWIN - 1) // _TWIN

        def wnd(g, _):
            def vec(v, _):
                t = tstart + g * _TWIN + v * 16 + lanes
                r = jnp.zeros((16,), jnp.int32)
                for sz in (128, 64, 32, 16, 8, 4, 2, 1):
                    c = r + sz
                    gc = plsc.load_gather(gv, [jnp.minimum(c, 255)])
                    r = jnp.where((c < _W * _NWIN) & (gc <= t), c, r)
                g0 = plsc.load_gather(gv, [r])
                s0 = plsc.load_gather(sv, [r])
                src = s0 + _CK * (t - g0)
                tabbuf[_v16(v)] = jnp.clip(src, 0, _H1N - _CK)
                return _
            nv = (jnp.minimum(n - g * _TWIN, _TWIN) + 15) // 16
            lax.fori_loop(0, nv, vec, None)
            toff = pl.multiple_of(tstart + g * _TWIN, 8)
            pltpu.sync_copy(tabbuf, tab_hbm.at[pl.ds(toff, _TWIN)])
            return _
        lax.fori_loop(0, nw, wnd, None)
        return _
    lax.fori_loop(0, _BINS_PER_W, per_bin, None)


_tabgen = functools.partial(
    pl.kernel,
    out_type=jax.ShapeDtypeStruct((_TABN,), jnp.int32),
    mesh=_mesh,
    compiler_params=_SC_PARAMS,
    scratch_types=[pltpu.VMEM((_W * _NWIN,), jnp.int32),
                   pltpu.VMEM((_W * _NWIN,), jnp.int32),
                   pltpu.VMEM((_NBINS,), jnp.int32),
                   pltpu.VMEM((_NBINS,), jnp.int32),
                   pltpu.VMEM((_TWIN,), jnp.int32)],
)(_tabgen_body)


def _apply_body(h1i_hbm, h1v_hbm, chtab_hbm, grid_hbm, chs_hbm, nch_hbm,
                grid_out, part_out, chunk, lbiA, lbvA, chbufA, lbiB, lbvB,
                chbufB, chs_v, nch_v, accv, semA, semB):
    w = _wid()
    lanes = _lanes()
    pltpu.sync_copy(chs_hbm, chs_v)
    pltpu.sync_copy(nch_hbm, nch_v)
    accv[...] = jnp.zeros((16,), jnp.float32)

    def per_bin(i, _):
        b = i * _W + w
        chstart = _scal(chs_v, b)
        nch = _scal(nch_v, b)
        cbase = pl.multiple_of(b * _BINSZ, 8)
        pltpu.sync_copy(grid_hbm.at[pl.ds(cbase, _BINSZ)], chunk)

        def ema(v, _):
            for u in range(8):
                g = chunk[_v16(v * 8 + u)]
                chunk[_v16(v * 8 + u)] = jnp.where(g < 0.0, g, g * _DECAY)
            return _
        lax.fori_loop(0, _BINSZ // 128, ema, None)

        ngroups = (nch + _GRP - 1) // _GRP

        def issue_group(g, lbi, lbv, chbuf, sem):
            coff = pl.multiple_of(chstart + g * _GRP, 8)
            pltpu.sync_copy(chtab_hbm.at[pl.ds(coff, _GRP)], chbuf)
            nmin = jnp.minimum(nch - g * _GRP, _GRP)

            def issue(k, _):
                src = pl.multiple_of(_scal(chbuf, k), 8)
                dst = pl.multiple_of(k * _CK, 8)
                pltpu.make_async_copy(
                    h1i_hbm.at[pl.ds(src, _CK)],
                    lbi.at[pl.ds(dst, _CK)], sem).start()
                pltpu.make_async_copy(
                    h1v_hbm.at[pl.ds(src, _CK)],
                    lbv.at[pl.ds(dst, _CK)], sem).start()
                return _
            lax.fori_loop(0, nmin, issue, None)

        def wait_group(g, lbi, lbv, chbuf, sem):
            nmin = jnp.minimum(nch - g * _GRP, _GRP)

            @pl.when(nmin == _GRP)
            def _bulk_drain():
                pltpu.make_async_copy(
                    h1i_hbm.at[pl.ds(0, _GRP * _CK)], lbi, sem).wait()
                pltpu.make_async_copy(
                    h1v_hbm.at[pl.ds(0, _GRP * _CK)], lbv, sem).wait()

            @pl.when(nmin < _GRP)
            def _tail_drain():
                def drain(k, _):
                    src = pl.multiple_of(_scal(chbuf, k), 8)
                    dst = pl.multiple_of(k * _CK, 8)
                    pltpu.make_async_copy(
                        h1i_hbm.at[pl.ds(src, _CK)],
                        lbi.at[pl.ds(dst, _CK)], sem).wait()
                    pltpu.make_async_copy(
                        h1v_hbm.at[pl.ds(src, _CK)],
                        lbv.at[pl.ds(dst, _CK)], sem).wait()
                    return _
                lax.fori_loop(0, nmin, drain, None)

        def process_group(g, lbi, lbv):
            vcnt = jnp.minimum(nch - g * _GRP, _GRP) * _CK

            def vec(v, _):
                pos = v * 16 + lanes
                iv = lbi[_v16(v)]
                vv = lbv[_v16(v)] * _MS
                inb = (pos < vcnt) & (iv >= cbase) & (iv < cbase + _BINSZ)
                iv = jnp.where(inb, iv, _SENT)
                vv = jnp.where(inb, vv, -1.0)
                si, sv = plsc.sort_key_val(iv, vv)
                nxt = _gather16(si, jnp.minimum(lanes + 1, 15))
                anydup = jnp.any((si == nxt) & (lanes < 15))

                def segmax(sv):
                    for s in (1, 2, 4, 8):
                        pi = _gather16(si, jnp.maximum(lanes - s, 0))
                        pv = _gather16(sv, jnp.maximum(lanes - s, 0))
                        take = (pi == si) & (lanes >= s)
                        sv = jnp.where(take, jnp.maximum(sv, pv), sv)
                    return sv
                sv = lax.cond(anydup, segmax, lambda x: x, sv)
                last = (si != nxt) | (lanes == 15)
                valid = si != _SENT
                local = jnp.clip(si - cbase, 0, _BINSZ - 1)
                wm = last & valid
                cur = plsc.load_gather(chunk, [local], mask=wm)
                upd = jnp.maximum(cur, sv)
                wm = wm & (cur >= 0.0)
                plsc.store_scatter(chunk, [local], upd, mask=wm)
                return _
            lax.fori_loop(0, (vcnt + 15) // 16, vec, None)

        @pl.when(ngroups > 0)
        def _prologue():
            issue_group(0, lbiA, lbvA, chbufA, semA)

        def pairloop(j, _):
            g0 = j * 2
            wait_group(g0, lbiA, lbvA, chbufA, semA)

            @pl.when(g0 + 1 < ngroups)
            def _issue_b():
                issue_group(g0 + 1, lbiB, lbvB, chbufB, semB)
            process_group(g0, lbiA, lbvA)

            @pl.when(g0 + 1 < ngroups)
            def _second():
                wait_group(g0 + 1, lbiB, lbvB, chbufB, semB)

                @pl.when(g0 + 2 < ngroups)
                def _issue_a():
                    issue_group(g0 + 2, lbiA, lbvA, chbufA, semA)
                process_group(g0 + 1, lbiB, lbvB)
            return _
        lax.fori_loop(0, (ngroups + 1) // 2, pairloop, None)

        pltpu.sync_copy(chunk, grid_out.at[pl.ds(cbase, _BINSZ)])

        @pl.when(b < _NE_LVL // _BINSZ)
        def _mean():
            def acc(v, a):
                for u in range(8):
                    a = a + jnp.maximum(chunk[_v16(v * 8 + u)], 0.0)
                return a
            accv[...] = accv[...] + lax.fori_loop(
                0, _BINSZ // 128, acc, jnp.zeros((16,), jnp.float32))
        return _
    lax.fori_loop(0, _BINS_PER_W, per_bin, None)
    pltpu.sync_copy(accv, part_out.at[w])


_apply = functools.partial(
    pl.kernel,
    out_type=(jax.ShapeDtypeStruct((_NE,), jnp.float32),
              jax.ShapeDtypeStruct((_W, 16), jnp.float32)),
    mesh=_mesh,
    compiler_params=_SC_PARAMS,
    scratch_types=[pltpu.VMEM((_BINSZ,), jnp.float32),
                   pltpu.VMEM((_GRP * _CK,), jnp.int32),
                   pltpu.VMEM((_GRP * _CK,), jnp.float32),
                   pltpu.VMEM((_GRP,), jnp.int32),
                   pltpu.VMEM((_GRP * _CK,), jnp.int32),
                   pltpu.VMEM((_GRP * _CK,), jnp.float32),
                   pltpu.VMEM((_GRP,), jnp.int32),
                   pltpu.VMEM((_NBINS,), jnp.int32),
                   pltpu.VMEM((_NBINS,), jnp.int32),
                   pltpu.VMEM((16,), jnp.float32),
                   pltpu.SemaphoreType.DMA,
                   pltpu.SemaphoreType.DMA],
)(_apply_body)


def _bitfield_body(thres_ref, g_ref, out_ref):
    t = thres_ref[0, 0]
    x = g_ref[...]
    bits = (x > t).astype(jnp.float32)
    l = lax.broadcasted_iota(jnp.int32, (128, 16), 0)
    k = lax.broadcasted_iota(jnp.int32, (128, 16), 1)
    w = jnp.where(l // 8 == k, jnp.exp2((l % 8).astype(jnp.float32)), 0.0)
    packed = lax.dot_general(bits, w, (((1,), (0,)), ((), ())),
                             preferred_element_type=jnp.float32)
    out_ref[...] = packed.astype(jnp.uint8)


def _bitfield(new_grid, thres):
    BLK = 2048
    rows = _NE // 128
    nblk = rows // BLK
    g2 = new_grid.reshape(rows, 128)
    out = pl.pallas_call(
        _bitfield_body,
        grid=(nblk,),
        in_specs=[pl.BlockSpec(memory_space=pltpu.SMEM),
                  pl.BlockSpec((BLK, 128), lambda i: (i, 0))],
        out_specs=pl.BlockSpec((BLK, 16), lambda i: (i, 0)),
        out_shape=jax.ShapeDtypeStruct((rows, 16), jnp.uint8),
    )(thres.reshape(1, 1), g2)
    return out.reshape(-1)


def kernel(density, idx_sample, density_grid):
    h1i, h1v, counts = _route(idx_sample, density)

    c = counts.reshape(_W, _NWIN, _NBINS)
    padded = (c + 7) & ~7
    instart = jnp.cumsum(padded, axis=2) - padded
    base = ((jnp.arange(_W, dtype=jnp.int32)[:, None, None] * _NWIN
             + jnp.arange(_NWIN, dtype=jnp.int32)[None, :, None]) * _WSP)
    srcstart = base + instart
    p_t = padded.transpose(2, 0, 1).reshape(_NBINS, _W * _NWIN)
    s_t = srcstart.transpose(2, 0, 1).reshape(_NBINS, _W * _NWIN)
    L = (p_t + _CK - 1) // _CK
    nch = L.sum(axis=1).astype(jnp.int32)
    nchr = (nch + _TWIN - 1) // _TWIN * _TWIN
    chstart = (jnp.cumsum(nchr) - nchr).astype(jnp.int32)
    rp = (jnp.cumsum(L, axis=1) - L).astype(jnp.int32)
    gt = chstart[:, None] + rp
    chtab = _tabgen(gt, s_t.astype(jnp.int32), chstart, nch)

    new_grid, partials = _apply(h1i, h1v, chtab, density_grid, chstart, nch)
    mean = partials.sum() / jnp.float32(_NE_LVL)
    thres = jnp.minimum(jnp.float32(_OPA), mean)
    return new_grid, _bitfield(new_grid, thres)
